# Initial kernel scaffold; baseline (speedup 1.0000x reference)
import jax
import jax.numpy as jnp
from jax.experimental import pallas as pl


def kernel(shelf_embs, product_embs, supply, W_q, b_q, W_k, b_k, W_v, b_v, W_o, b_o):
    raise NotImplementedError("write your pallas kernel here")


# speedup vs baseline: 302.2700x; 302.2700x over previous
import functools

import jax
import jax.numpy as jnp
from jax import lax
from jax.experimental import pallas as pl

B, S, P = 2, 128, 256
D = 1024
H = 16
DH = D // H


def _attn_body(shelf_ref, product_ref, supply_ref, wq_ref, bq_ref,
               wkv_ref, bkv_ref, attn_ref):
    x_s = shelf_ref[0]
    x_p = product_ref[0]

    q = jnp.dot(x_s, wq_ref[...], preferred_element_type=jnp.float32) + bq_ref[...]
    kv = jnp.dot(x_p, wkv_ref[...], preferred_element_type=jnp.float32) + bkv_ref[...]
    k = kv[:, :D]
    v = kv[:, D:]

    q4 = q.reshape(S, H, DH)
    k4 = k.reshape(P, H, DH)
    v4 = v.reshape(P, H, DH)

    logits = lax.dot_general(
        q4, k4,
        dimension_numbers=(((2,), (2,)), ((1,), (1,))),
        preferred_element_type=jnp.float32,
    ) * (1.0 / (DH ** 0.5))

    mask = (supply_ref[0] > 0)[None, :, :]
    masked = jnp.where(mask, logits, -1e30)
    m = jnp.max(masked, axis=2, keepdims=True)
    e = jnp.where(mask, jnp.exp(logits - m), 0.0)
    den = jnp.sum(e, axis=2, keepdims=True)
    w = e / (den + 1e-9)

    attn_ref[0] = lax.dot_general(
        w, v4,
        dimension_numbers=(((2,), (0,)), ((0,), (1,))),
        preferred_element_type=jnp.float32,
    )


def _oproj_body(x_ref, wo_ref, bo_ref, out_ref):
    out_ref[...] = jnp.dot(x_ref[...], wo_ref[...],
                           preferred_element_type=jnp.float32) + bo_ref[...]


@jax.jit
def kernel(shelf_embs, product_embs, supply, W_q, b_q, W_k, b_k, W_v, b_v, W_o, b_o):
    wq_t = W_q.T
    wkv_t = jnp.concatenate([W_k.T, W_v.T], axis=1)
    bkv = jnp.concatenate([b_k, b_v])
    wo_t = W_o.T

    attn = pl.pallas_call(
        _attn_body,
        grid=(B,),
        in_specs=[
            pl.BlockSpec((1, S, D), lambda b: (b, 0, 0)),
            pl.BlockSpec((1, P, D), lambda b: (b, 0, 0)),
            pl.BlockSpec((1, S, P), lambda b: (b, 0, 0)),
            pl.BlockSpec((D, D), lambda b: (0, 0)),
            pl.BlockSpec((D,), lambda b: (0,)),
            pl.BlockSpec((D, 2 * D), lambda b: (0, 0)),
            pl.BlockSpec((2 * D,), lambda b: (0,)),
        ],
        out_specs=pl.BlockSpec((1, H, S, DH), lambda b: (b, 0, 0, 0)),
        out_shape=jax.ShapeDtypeStruct((B, H, S, DH), jnp.float32),
    )(shelf_embs, product_embs, supply, wq_t, b_q, wkv_t, bkv)

    scr = attn.reshape(B * S, D)

    out = pl.pallas_call(
        _oproj_body,
        out_shape=jax.ShapeDtypeStruct((B * S, D), jnp.float32),
    )(scr, wo_t, b_o)
    return out.reshape(B, S, D)


# speedup vs baseline: 668.4914x; 2.2116x over previous
import jax
import jax.numpy as jnp
from jax import lax
from jax.experimental import pallas as pl

B, S, P = 2, 128, 256
D = 1024
H = 16
DH = D // H

_XWT = (((1,), (1,)), ((), ()))


def _attn_body(shelf_ref, product_ref, supply_ref, wq_ref, bq_ref,
               wk_ref, bk_ref, wv_ref, bv_ref, attn_ref):
    x_s = shelf_ref[0]
    x_p = product_ref[0]

    q = lax.dot_general(x_s, wq_ref[...], _XWT,
                        preferred_element_type=jnp.float32) + bq_ref[...]
    k = lax.dot_general(x_p, wk_ref[...], _XWT,
                        preferred_element_type=jnp.float32) + bk_ref[...]
    v = lax.dot_general(x_p, wv_ref[...], _XWT,
                        preferred_element_type=jnp.float32) + bv_ref[...]

    q4 = q.reshape(S, H, DH)
    k4 = k.reshape(P, H, DH)
    v4 = v.reshape(P, H, DH)

    logits = lax.dot_general(
        q4, k4,
        dimension_numbers=(((2,), (2,)), ((1,), (1,))),
        preferred_element_type=jnp.float32,
    ) * (1.0 / (DH ** 0.5))

    mask = (supply_ref[0] > 0)[None, :, :]
    masked = jnp.where(mask, logits, -1e30)
    m = jnp.max(masked, axis=2, keepdims=True)
    e = jnp.where(mask, jnp.exp(logits - m), 0.0)
    den = jnp.sum(e, axis=2, keepdims=True)
    w = e / (den + 1e-9)

    attn_ref[0] = lax.dot_general(
        w, v4,
        dimension_numbers=(((2,), (0,)), ((0,), (1,))),
        preferred_element_type=jnp.float32,
    )


def _oproj_body(x_ref, wo_ref, bo_ref, out_ref):
    out_ref[...] = lax.dot_general(x_ref[...], wo_ref[...], _XWT,
                                   preferred_element_type=jnp.float32) + bo_ref[...]


@jax.jit
def kernel(shelf_embs, product_embs, supply, W_q, b_q, W_k, b_k, W_v, b_v, W_o, b_o):
    attn = pl.pallas_call(
        _attn_body,
        grid=(B,),
        in_specs=[
            pl.BlockSpec((1, S, D), lambda b: (b, 0, 0)),
            pl.BlockSpec((1, P, D), lambda b: (b, 0, 0)),
            pl.BlockSpec((1, S, P), lambda b: (b, 0, 0)),
            pl.BlockSpec((D, D), lambda b: (0, 0)),
            pl.BlockSpec((D,), lambda b: (0,)),
            pl.BlockSpec((D, D), lambda b: (0, 0)),
            pl.BlockSpec((D,), lambda b: (0,)),
            pl.BlockSpec((D, D), lambda b: (0, 0)),
            pl.BlockSpec((D,), lambda b: (0,)),
        ],
        out_specs=pl.BlockSpec((1, H, S, DH), lambda b: (b, 0, 0, 0)),
        out_shape=jax.ShapeDtypeStruct((B, H, S, DH), jnp.float32),
    )(shelf_embs, product_embs, supply, W_q, b_q, W_k, b_k, W_v, b_v)

    scr = attn.reshape(B * S, D)

    out = pl.pallas_call(
        _oproj_body,
        out_shape=jax.ShapeDtypeStruct((B * S, D), jnp.float32),
    )(scr, W_o, b_o)
    return out.reshape(B, S, D)
